# baseline (speedup 1.0000x reference)
import functools

import jax
import jax.numpy as jnp
from jax import lax
from jax.experimental import pallas as pl
from jax.experimental.pallas import tpu as pltpu
from jax.experimental.pallas import tpu_sc as plsc

D = 128
NF = 4
LANES = 16
NC, NS = 2, 16
NW = NC * NS


def _prep_body(table_ref, wf_ref, bfc_ref, wc_ref, tbl_out_ref, wfc_out_ref):
    wc1 = wc_ref[:D, :]
    wc2 = wc_ref[D:, :]
    bias = (
        jnp.dot(bfc_ref[0:1, :], wc2, preferred_element_type=jnp.float32)
        + bfc_ref[1:2, :]
    )
    tbl_out_ref[...] = (
        jnp.dot(table_ref[...], wc1, preferred_element_type=jnp.float32) + bias
    )
    wfc_out_ref[...] = jnp.dot(wf_ref[...], wc2, preferred_element_type=jnp.float32)


def _prep(table, Wf, bfc, Wc):
    num_atoms = table.shape[0]
    return pl.pallas_call(
        _prep_body,
        out_shape=[
            jax.ShapeDtypeStruct((num_atoms, D), jnp.float32),
            jax.ShapeDtypeStruct((NF, D), jnp.float32),
        ],
    )(table, Wf, bfc, Wc)


@functools.partial(jax.jit, static_argnames=("num_atoms", "batch", "seq"))
def _sc_lookup(ids, feat, tbl, wfc, *, num_atoms, batch, seq):
    n_rows = batch * seq
    rows_per_w = n_rows // NW
    b_per_w = batch // NW
    half = seq // (2 * LANES) * LANES
    n_fullg = half // LANES
    tail_l0 = seq - LANES if seq > 2 * half else None
    mesh = plsc.VectorSubcoreMesh(
        core_axis_name="c", subcore_axis_name="s", num_cores=NC, num_subcores=NS
    )

    @functools.partial(
        pl.kernel,
        out_type=jax.ShapeDtypeStruct((n_rows, D), jnp.float32),
        mesh=mesh,
        compiler_params=pltpu.CompilerParams(needs_layout_passes=False),
        scratch_types=[
            pltpu.VMEM((num_atoms * D,), jnp.float32),
            pltpu.VMEM((NF, D), jnp.float32),
            pltpu.VMEM((rows_per_w // D, D), jnp.int32),
            pltpu.VMEM((half, NF, b_per_w), jnp.float32),
            pltpu.VMEM((half, D), jnp.float32),
            pltpu.VMEM((half, D), jnp.float32),
            pltpu.SemaphoreType.DMA,
        ],
    )
    def body(ids_hbm, feat_hbm, tbl_hbm, wfc_hbm, out_hbm, tbl_v, wfc_v, ids_v, feat_v, out_a, out_b, sem):
        wid = lax.axis_index("s") * NC + lax.axis_index("c")
        base0 = wid * rows_per_w
        pltpu.sync_copy(tbl_hbm, tbl_v)
        pltpu.sync_copy(wfc_hbm, wfc_v)
        pltpu.sync_copy(
            ids_hbm.at[pl.ds(pl.multiple_of(base0 // D, 8), rows_per_w // D)], ids_v
        )
        cols = [lax.iota(jnp.int32, LANES) + LANES * j for j in range(D // LANES)]
        iota = lax.iota(jnp.int32, LANES)
        w = [
            [wfc_v[k, pl.ds(LANES * j, LANES)] for j in range(D // LANES)]
            for k in range(NF)
        ]
        wp = [
            [
                plsc.pack(w[k][2 * jj], w[k][2 * jj + 1],
                          format=plsc.PackFormat.INTERLEAVED)
                for jj in range(D // (2 * LANES))
            ]
            for k in range(NF)
        ]

        tbl_slices = [
            tbl_v.at[pl.ds(LANES * j, (num_atoms - 1) * D + LANES)]
            for j in range(D // LANES)
        ]

        def do_group(bb, l0, lrel0, out_v):
            r0 = bb * seq + l0 + lrel0
            rvec = r0 + iota
            idbase = plsc.load_gather(ids_v, [rvec // D, rvec % D]) * D
            lvec = lrel0 + iota
            fk = [
                plsc.load_gather(
                    feat_v,
                    [lvec, jnp.full((LANES,), q, jnp.int32),
                     jnp.full((LANES,), bb, jnp.int32)],
                )
                for q in range(NF)
            ]
            for rr in range(LANES):
                fp = []
                for q in range(NF):
                    fv = jnp.full((LANES,), fk[q][rr], jnp.float32)
                    fp.append(
                        plsc.pack(fv, fv, format=plsc.PackFormat.INTERLEAVED)
                    )
                idx = idbase[rr] + iota
                for jj in range(D // (2 * LANES)):
                    s = (fp[0] * wp[0][jj] + fp[1] * wp[1][jj]) + (
                        fp[2] * wp[2][jj] + fp[3] * wp[3][jj]
                    )
                    s0, s1 = plsc.unpack(s, format=plsc.PackFormat.INTERLEAVED)
                    g0 = plsc.load_gather(tbl_slices[2 * jj], [idx])
                    g1 = plsc.load_gather(tbl_slices[2 * jj + 1], [idx])
                    out_v[lrel0 + rr, pl.ds(LANES * 2 * jj, LANES)] = g0 + s0
                    out_v[lrel0 + rr, pl.ds(LANES * (2 * jj + 1), LANES)] = g1 + s1

        def out_slice(bb, l0, hs):
            return out_hbm.at[
                pl.ds(pl.multiple_of(base0 + bb * seq + l0, 8), hs)
            ]

        def issue(bb, l0, out_v, hs):
            pltpu.async_copy(out_v.at[pl.ds(0, hs)], out_slice(bb, l0, hs), sem)

        def wait_one(l0, hs):
            pltpu.make_async_copy(
                out_a.at[pl.ds(0, hs)], out_slice(0, l0, hs), sem
            ).wait()

        def run_piece(l0, compute, hs):
            compute(0, out_a)
            issue(0, l0, out_a, hs)
            compute(1, out_b)
            issue(1, l0, out_b, hs)

            def pair_body(p, c):
                bb = 2 * p
                wait_one(l0, hs)
                compute(bb, out_a)
                issue(bb, l0, out_a, hs)
                wait_one(l0, hs)
                compute(bb + 1, out_b)
                issue(bb + 1, l0, out_b, hs)
                return c

            lax.fori_loop(1, b_per_w // 2, pair_body, 0)
            wait_one(l0, hs)
            wait_one(l0, hs)

        def l_body(li, carry):
            l0 = li * half
            pltpu.sync_copy(
                feat_hbm.at[
                    pl.ds(pl.multiple_of(l0, 8), half),
                    slice(None),
                    pl.ds(wid * b_per_w, b_per_w),
                ],
                feat_v,
            )

            def compute(bb, out_v):
                @plsc.parallel_loop(0, n_fullg, unroll=3)
                def group_body(g):
                    do_group(bb, l0, g * LANES, out_v)

            run_piece(l0, compute, half)
            return carry

        lax.fori_loop(0, 2, l_body, 0)

        if tail_l0 is not None:
            pltpu.sync_copy(
                feat_hbm.at[
                    pl.ds(tail_l0, LANES),
                    slice(None),
                    pl.ds(wid * b_per_w, b_per_w),
                ],
                feat_v.at[pl.ds(0, LANES)],
            )

            def compute_tail(bb, out_v):
                do_group(bb, tail_l0, 0, out_v)

            run_piece(tail_l0, compute_tail, LANES)

    return body(ids, feat, tbl, wfc)


def kernel(atom_ids, atom_features, table, Wf, bf, Wc, bc):
    B, L = atom_ids.shape
    n_rows = B * L
    ids = atom_ids.reshape(n_rows // D, D).astype(jnp.int32)
    feat = atom_features.transpose(1, 2, 0)
    bfc = jnp.stack([bf, bc])
    tbl, wfc = _prep(table, Wf, bfc, Wc)
    out = _sc_lookup(
        ids, feat, tbl.reshape(-1), wfc,
        num_atoms=table.shape[0], batch=B, seq=L,
    )
    return out.reshape(B, L, D)


# speedup vs baseline: 2.0327x; 2.0327x over previous
import functools

import jax
import jax.numpy as jnp
from jax import lax
from jax.experimental import pallas as pl
from jax.experimental.pallas import tpu as pltpu
from jax.experimental.pallas import tpu_sc as plsc

D = 128
NF = 4
LANES = 16
NC, NS = 2, 16
NW = NC * NS


def _prep_body(table_ref, wf_ref, bfc_ref, wc_ref, tbl_out_ref, wfc_out_ref):
    wc1 = wc_ref[:D, :]
    wc2 = wc_ref[D:, :]
    bias = (
        jnp.dot(bfc_ref[0:1, :], wc2, preferred_element_type=jnp.float32)
        + bfc_ref[1:2, :]
    )
    tbl_out_ref[...] = (
        jnp.dot(table_ref[...], wc1, preferred_element_type=jnp.float32) + bias
    )
    wfc_out_ref[...] = jnp.dot(wf_ref[...], wc2, preferred_element_type=jnp.float32)


def _prep(table, Wf, bfc, Wc):
    num_atoms = table.shape[0]
    return pl.pallas_call(
        _prep_body,
        out_shape=[
            jax.ShapeDtypeStruct((num_atoms, D), jnp.float32),
            jax.ShapeDtypeStruct((NF, D), jnp.float32),
        ],
    )(table, Wf, bfc, Wc)


@functools.partial(jax.jit, static_argnames=("num_atoms", "batch", "seq"))
def _sc_lookup(ids, feat, tbl, wfc, *, num_atoms, batch, seq):
    n_rows = batch * seq
    rows_per_w = n_rows // NW
    b_per_w = batch // NW
    half = seq // (2 * LANES) * LANES
    n_fullg = half // LANES
    tail_l0 = seq - LANES if seq > 2 * half else None
    mesh = plsc.VectorSubcoreMesh(
        core_axis_name="c", subcore_axis_name="s", num_cores=NC, num_subcores=NS
    )

    @functools.partial(
        pl.kernel,
        out_type=jax.ShapeDtypeStruct((n_rows, D), jnp.float32),
        mesh=mesh,
        compiler_params=pltpu.CompilerParams(needs_layout_passes=False),
        scratch_types=[
            pltpu.VMEM((num_atoms * D,), jnp.float32),
            pltpu.VMEM((NF, D), jnp.float32),
            pltpu.VMEM((rows_per_w // D, D), jnp.int32),
            pltpu.VMEM((half, NF, b_per_w), jnp.float32),
            pltpu.VMEM((half, D), jnp.float32),
            pltpu.VMEM((half, D), jnp.float32),
            pltpu.SemaphoreType.DMA,
        ],
    )
    def body(ids_hbm, feat_hbm, tbl_hbm, wfc_hbm, out_hbm, tbl_v, wfc_v, ids_v, feat_v, out_a, out_b, sem):
        wid = lax.axis_index("s") * NC + lax.axis_index("c")
        base0 = wid * rows_per_w
        pltpu.sync_copy(tbl_hbm, tbl_v)
        pltpu.sync_copy(wfc_hbm, wfc_v)
        pltpu.sync_copy(
            ids_hbm.at[pl.ds(pl.multiple_of(base0 // D, 8), rows_per_w // D)], ids_v
        )
        cols = [lax.iota(jnp.int32, LANES) + LANES * j for j in range(D // LANES)]
        iota = lax.iota(jnp.int32, LANES)
        w = [
            [wfc_v[k, pl.ds(LANES * j, LANES)] for j in range(D // LANES)]
            for k in range(NF)
        ]
        wp = [
            [
                plsc.pack(w[k][2 * jj], w[k][2 * jj + 1],
                          format=plsc.PackFormat.INTERLEAVED)
                for jj in range(D // (2 * LANES))
            ]
            for k in range(NF)
        ]

        tbl_slices = [
            tbl_v.at[pl.ds(LANES * j, (num_atoms - 1) * D + LANES)]
            for j in range(D // LANES)
        ]

        def do_group(bb, l0, lrel0, out_v):
            r0 = bb * seq + l0 + lrel0
            rvec = r0 + iota
            idbase = plsc.load_gather(ids_v, [rvec // D, rvec % D]) * D
            lvec = lrel0 + iota
            fk = [
                plsc.load_gather(
                    feat_v,
                    [lvec, jnp.full((LANES,), q, jnp.int32),
                     jnp.full((LANES,), bb, jnp.int32)],
                )
                for q in range(NF)
            ]
            for rr in range(LANES):
                fp = []
                for q in range(NF):
                    fv = jnp.full((LANES,), fk[q][rr], jnp.float32)
                    fp.append(
                        plsc.pack(fv, fv, format=plsc.PackFormat.INTERLEAVED)
                    )
                idx = idbase[rr] + iota
                for jj in range(D // (2 * LANES)):
                    s = (fp[0] * wp[0][jj] + fp[1] * wp[1][jj]) + (
                        fp[2] * wp[2][jj] + fp[3] * wp[3][jj]
                    )
                    s0, s1 = plsc.unpack(s, format=plsc.PackFormat.INTERLEAVED)
                    g0 = plsc.load_gather(tbl_slices[2 * jj], [idx])
                    g1 = plsc.load_gather(tbl_slices[2 * jj + 1], [idx])
                    out_v[lrel0 + rr, pl.ds(LANES * 2 * jj, LANES)] = g0 + s0
                    out_v[lrel0 + rr, pl.ds(LANES * (2 * jj + 1), LANES)] = g1 + s1

        def out_slice(bb, l0, hs):
            return out_hbm.at[
                pl.ds(pl.multiple_of(base0 + bb * seq + l0, 8), hs)
            ]

        def issue(bb, l0, out_v, hs):
            pltpu.async_copy(out_v.at[pl.ds(0, hs)], out_slice(bb, l0, hs), sem)

        def wait_one(l0, hs):
            pltpu.make_async_copy(
                out_a.at[pl.ds(0, hs)], out_slice(0, l0, hs), sem
            ).wait()

        def run_piece(l0, compute, hs):
            compute(0, out_a)
            issue(0, l0, out_a, hs)
            compute(1, out_b)
            issue(1, l0, out_b, hs)

            def pair_body(p, c):
                bb = 2 * p
                wait_one(l0, hs)
                compute(bb, out_a)
                issue(bb, l0, out_a, hs)
                wait_one(l0, hs)
                compute(bb + 1, out_b)
                issue(bb + 1, l0, out_b, hs)
                return c

            lax.fori_loop(1, b_per_w // 2, pair_body, 0)
            wait_one(l0, hs)
            wait_one(l0, hs)

        def l_body(li, carry):
            l0 = li * half
            pltpu.sync_copy(
                feat_hbm.at[
                    pl.ds(pl.multiple_of(l0, 8), half),
                    slice(None),
                    pl.ds(wid * b_per_w, b_per_w),
                ],
                feat_v,
            )

            def compute(bb, out_v):
                @plsc.parallel_loop(0, n_fullg, unroll=2)
                def group_body(g):
                    do_group(bb, l0, g * LANES, out_v)

            run_piece(l0, compute, half)
            return carry

        lax.fori_loop(0, 2, l_body, 0)

        if tail_l0 is not None:
            pltpu.sync_copy(
                feat_hbm.at[
                    pl.ds(tail_l0, LANES),
                    slice(None),
                    pl.ds(wid * b_per_w, b_per_w),
                ],
                feat_v.at[pl.ds(0, LANES)],
            )

            def compute_tail(bb, out_v):
                do_group(bb, tail_l0, 0, out_v)

            run_piece(tail_l0, compute_tail, LANES)

    return body(ids, feat, tbl, wfc)


def kernel(atom_ids, atom_features, table, Wf, bf, Wc, bc):
    B, L = atom_ids.shape
    n_rows = B * L
    ids = atom_ids.reshape(n_rows // D, D).astype(jnp.int32)
    feat = atom_features.transpose(1, 2, 0)
    bfc = jnp.stack([bf, bc])
    tbl, wfc = _prep(table, Wf, bfc, Wc)
    out = _sc_lookup(
        ids, feat, tbl.reshape(-1), wfc,
        num_atoms=table.shape[0], batch=B, seq=L,
    )
    return out.reshape(B, L, D)
